# Initial kernel scaffold; baseline (speedup 1.0000x reference)
import jax
import jax.numpy as jnp
from jax.experimental import pallas as pl


def kernel(X, initial_states, A, B, C):
    raise NotImplementedError("write your pallas kernel here")


# speedup vs baseline: 2.3785x; 2.3785x over previous
import jax
import jax.numpy as jnp
from jax import lax
from jax.experimental import pallas as pl
from jax.experimental.pallas import tpu as pltpu

_L = 256
_NC = 4096 // _L


def _ssd_kernel(x_ref, a_ref, b_ref, c_ref, init_ref, y_ref):
    xs = x_ref[0]
    bs = b_ref[0]
    cs = c_ref[0]
    av = a_ref[0]

    row = lax.broadcasted_iota(jnp.int32, (_L, _L), 0)
    col = lax.broadcasted_iota(jnp.int32, (_L, _L), 1)
    ltri = row >= col

    r = init_ref[0]

    for k in range(_NC):
        sl = slice(k * _L, (k + 1) * _L)
        x = xs[sl, :]
        b = bs[sl, :]
        c = cs[sl, :]
        a = av[:, sl]

        a_b = jnp.broadcast_to(a, (_L, _L))
        csum = jnp.sum(jnp.where(ltri, a_b, 0.0), axis=1, keepdims=True)
        a_last = jnp.sum(a, axis=1, keepdims=True)
        e_pos = jnp.exp(csum)
        e_neg = jnp.exp(-csum)

        b_sc = b * e_neg

        scores = lax.dot_general(
            c, b_sc, (((1,), (1,)), ((), ())),
            preferred_element_type=jnp.float32)
        scores = jnp.where(ltri, scores, 0.0)

        y_diag = jnp.dot(scores, x, preferred_element_type=jnp.float32)
        y_off = lax.dot_general(
            c, r, (((1,), (1,)), ((), ())),
            preferred_element_type=jnp.float32)
        y_ref[0, sl, :] = e_pos * (y_diag + y_off)

        local = lax.dot_general(
            x, b_sc, (((0,), (0,)), ((), ())),
            preferred_element_type=jnp.float32)
        r = jnp.exp(a_last) * (r + local)


def kernel(X, initial_states, A, B, C):
    b, S, h, p = X.shape
    n = B.shape[-1]
    bh = b * h

    Xr = X.transpose(0, 2, 1, 3).reshape(bh, S, p)
    Br = B.transpose(0, 2, 1, 3).reshape(bh, S, n)
    Cr = C.transpose(0, 2, 1, 3).reshape(bh, S, n)
    Ar = A.transpose(0, 2, 1).reshape(bh, 1, S)
    Ir = initial_states.reshape(b, h, p, n).reshape(bh, p, n)

    Yr = pl.pallas_call(
        _ssd_kernel,
        out_shape=jax.ShapeDtypeStruct((bh, S, p), jnp.float32),
        grid=(bh,),
        in_specs=[
            pl.BlockSpec((1, S, p), lambda i: (i, 0, 0)),
            pl.BlockSpec((1, 1, S), lambda i: (i, 0, 0)),
            pl.BlockSpec((1, S, n), lambda i: (i, 0, 0)),
            pl.BlockSpec((1, S, n), lambda i: (i, 0, 0)),
            pl.BlockSpec((1, p, n), lambda i: (i, 0, 0)),
        ],
        out_specs=pl.BlockSpec((1, S, p), lambda i: (i, 0, 0)),
        compiler_params=pltpu.CompilerParams(
            dimension_semantics=("parallel",),
            vmem_limit_bytes=100 * 1024 * 1024,
        ),
    )(Xr, Ar, Br, Cr, Ir)

    return Yr.reshape(b, h, S, p).transpose(0, 2, 1, 3)
